# Initial kernel scaffold; baseline (speedup 1.0000x reference)
import jax
import jax.numpy as jnp
from jax.experimental import pallas as pl


def kernel(x, embed_weight, concepts):
    raise NotImplementedError("write your pallas kernel here")


# speedup vs baseline: 3.4104x; 3.4104x over previous
import functools

import jax
import jax.numpy as jnp
from jax import lax
from jax.experimental import pallas as pl
from jax.experimental.pallas import tpu as pltpu
from jax.experimental.pallas import tpu_sc as plsc

DIM = 64
NC = 2
NS = 16
NW = NC * NS
CHUNK = 128


@functools.cache
def _make_kernel(B: int):
    b_per_w = B // NW
    n_chunks = b_per_w // CHUNK
    mesh = plsc.VectorSubcoreMesh(core_axis_name="c", subcore_axis_name="s")

    @functools.partial(
        pl.kernel,
        mesh=mesh,
        compiler_params=pltpu.CompilerParams(use_tc_tiling_on_sc=False),
        out_type=jax.ShapeDtypeStruct((B, DIM), jnp.float32),
        scratch_types=[
            pltpu.VMEM((n_chunks, CHUNK), jnp.int32),
            pltpu.VMEM((CHUNK, DIM), jnp.float32),
            pltpu.SemaphoreType.DMA,
        ],
    )
    def k(x_hbm, table_hbm, out_hbm, idx_v, rows_v, sem):
        wid = lax.axis_index("s") * NC + lax.axis_index("c")
        base = wid * b_per_w
        pltpu.sync_copy(x_hbm.at[wid], idx_v)

        def body(c, carry):
            off = pl.multiple_of(base + c * CHUNK, 8)
            pltpu.async_copy(table_hbm.at[idx_v.at[c]], rows_v, sem).wait()
            pltpu.sync_copy(rows_v, out_hbm.at[pl.ds(off, CHUNK)])
            return carry

        lax.fori_loop(0, n_chunks, body, 0)

    return k


def kernel(x, embed_weight, concepts):
    batch, hist = x.shape
    B = batch * hist
    table = jnp.concatenate([embed_weight, concepts], axis=0)
    xw = x.reshape(NW, B // NW // CHUNK, CHUNK)
    out = _make_kernel(B)(xw, table)
    return out.reshape(batch, hist, DIM)


# speedup vs baseline: 4.0659x; 1.1922x over previous
import functools

import jax
import jax.numpy as jnp
from jax import lax
from jax.experimental import pallas as pl
from jax.experimental.pallas import tpu as pltpu
from jax.experimental.pallas import tpu_sc as plsc

DIM = 64
NC = 2
NS = 16
NW = NC * NS
CHUNK = 128
NBUF = 8


@functools.cache
def _make_kernel(B: int):
    b_per_w = B // NW
    n_chunks = b_per_w // CHUNK
    n_groups = n_chunks // NBUF
    assert n_chunks % NBUF == 0
    mesh = plsc.VectorSubcoreMesh(core_axis_name="c", subcore_axis_name="s")

    @functools.partial(
        pl.kernel,
        mesh=mesh,
        compiler_params=pltpu.CompilerParams(use_tc_tiling_on_sc=False),
        out_type=jax.ShapeDtypeStruct((B, DIM), jnp.float32),
        scratch_types=[
            pltpu.VMEM((n_chunks, CHUNK), jnp.int32),
            pltpu.VMEM((NBUF, CHUNK, DIM), jnp.float32),
            pltpu.SemaphoreType.DMA((NBUF,)),
            pltpu.SemaphoreType.DMA((NBUF,)),
        ],
    )
    def k(x_hbm, table_hbm, out_hbm, idx_v, rows_v, gsem, ssem):
        wid = lax.axis_index("s") * NC + lax.axis_index("c")
        base = wid * b_per_w
        pltpu.sync_copy(x_hbm.at[wid], idx_v)

        def start_gather(b, c):
            pltpu.async_copy(table_hbm.at[idx_v.at[c]], rows_v.at[b], gsem.at[b])

        def wait_gather(b, c):
            pltpu.make_async_copy(
                table_hbm.at[idx_v.at[c]], rows_v.at[b], gsem.at[b]
            ).wait()

        def start_scatter(b, c):
            off = pl.multiple_of(base + c * CHUNK, CHUNK)
            pltpu.async_copy(rows_v.at[b], out_hbm.at[pl.ds(off, CHUNK)], ssem.at[b])

        def wait_scatter(b, c):
            off = pl.multiple_of(base + c * CHUNK, CHUNK)
            pltpu.make_async_copy(
                rows_v.at[b], out_hbm.at[pl.ds(off, CHUNK)], ssem.at[b]
            ).wait()

        for b in range(NBUF):
            start_gather(b, b)

        def group_body(g, carry):
            c0 = g * NBUF
            for b in range(NBUF):
                wait_gather(b, c0 + b)
                start_scatter(b, c0 + b)
            for b in range(NBUF):
                wait_scatter(b, c0 + b)
                start_gather(b, c0 + NBUF + b)
            return carry

        lax.fori_loop(0, n_groups - 1, group_body, 0)

        c0 = (n_groups - 1) * NBUF
        for b in range(NBUF):
            wait_gather(b, c0 + b)
            start_scatter(b, c0 + b)
        for b in range(NBUF):
            wait_scatter(b, c0 + b)

    return k


def kernel(x, embed_weight, concepts):
    batch, hist = x.shape
    B = batch * hist
    table = jnp.concatenate([embed_weight, concepts], axis=0)
    xw = x.reshape(NW, B // NW // CHUNK, CHUNK)
    out = _make_kernel(B)(xw, table)
    return out.reshape(batch, hist, DIM)


# speedup vs baseline: 4.2184x; 1.0375x over previous
import functools

import jax
import jax.numpy as jnp
from jax import lax
from jax.experimental import pallas as pl
from jax.experimental.pallas import tpu as pltpu
from jax.experimental.pallas import tpu_sc as plsc

DIM = 64
L = 16
NC = 2
NS = 16
NW = NC * NS
CHUNK = 128
NBUF = 8
NGRP = CHUNK // L


@functools.cache
def _make_kernel(B: int, num_embeds: int):
    b_per_w = B // NW
    n_chunks = b_per_w // CHUNK
    n_groups = n_chunks // NBUF
    assert n_chunks % NBUF == 0
    mesh = plsc.VectorSubcoreMesh(core_axis_name="c", subcore_axis_name="s")

    @functools.partial(
        pl.kernel,
        mesh=mesh,
        compiler_params=pltpu.CompilerParams(
            use_tc_tiling_on_sc=False, needs_layout_passes=False
        ),
        out_type=jax.ShapeDtypeStruct((B, DIM), jnp.float32),
        scratch_types=[
            pltpu.VMEM((n_chunks, CHUNK), jnp.int32),
            pltpu.VMEM((NBUF, CHUNK), jnp.int32),
            pltpu.VMEM((NBUF, CHUNK, DIM), jnp.float32),
            pltpu.VMEM((1, DIM), jnp.float32),
            pltpu.SemaphoreType.DMA((NBUF,)),
            pltpu.SemaphoreType.DMA((NBUF,)),
        ],
    )
    def k(x_hbm, table_hbm, conc_hbm, out_hbm, idx_v, cidx_v, rows_v, conc_v,
          gsem, ssem):
        wid = lax.axis_index("s") * NC + lax.axis_index("c")
        base = wid * b_per_w
        pltpu.sync_copy(conc_hbm, conc_v)
        pltpu.sync_copy(x_hbm.at[wid], idx_v)

        def clamp(b, c):
            for j in range(NGRP):
                v = idx_v[c, pl.ds(j * L, L)]
                cidx_v[b, pl.ds(j * L, L)] = jnp.where(v == num_embeds, 0, v)

        def fixup(b, c):
            acc = jnp.zeros((L,), jnp.int32)
            for j in range(NGRP):
                v = idx_v[c, pl.ds(j * L, L)]
                acc = acc + jnp.where(v == num_embeds, 1, 0)
            cnt = jnp.sum(acc)

            @pl.when(cnt > 0)
            def _():
                def grp_body(j, carry):
                    v = idx_v[c, pl.ds(j * L, L)]
                    mi = jnp.where(v == num_embeds, 1, 0)
                    gcnt = jnp.sum(mi)

                    @pl.when(gcnt > 0)
                    def _():
                        for l in range(L):
                            @pl.when(mi[l] > 0)
                            def _():
                                row = j * L + l
                                for q in range(DIM // L):
                                    rows_v[b, row, pl.ds(q * L, L)] = (
                                        conc_v[0, pl.ds(q * L, L)]
                                    )
                    return carry

                lax.fori_loop(0, NGRP, grp_body, 0)

        def start_gather(b):
            pltpu.async_copy(table_hbm.at[cidx_v.at[b]], rows_v.at[b], gsem.at[b])

        def wait_gather(b):
            pltpu.make_async_copy(
                table_hbm.at[cidx_v.at[b]], rows_v.at[b], gsem.at[b]
            ).wait()

        def start_scatter(b, c):
            off = pl.multiple_of(base + c * CHUNK, CHUNK)
            pltpu.async_copy(rows_v.at[b], out_hbm.at[pl.ds(off, CHUNK)], ssem.at[b])

        def wait_scatter(b, c):
            off = pl.multiple_of(base + c * CHUNK, CHUNK)
            pltpu.make_async_copy(
                rows_v.at[b], out_hbm.at[pl.ds(off, CHUNK)], ssem.at[b]
            ).wait()

        for b in range(NBUF):
            clamp(b, b)
            start_gather(b)

        def group_body(g, carry):
            c0 = g * NBUF
            for b in range(NBUF):
                wait_gather(b)
                fixup(b, c0 + b)
                start_scatter(b, c0 + b)
            for b in range(NBUF):
                wait_scatter(b, c0 + b)
                clamp(b, c0 + NBUF + b)
                start_gather(b)
            return carry

        lax.fori_loop(0, n_groups - 1, group_body, 0)

        c0 = (n_groups - 1) * NBUF
        for b in range(NBUF):
            wait_gather(b)
            fixup(b, c0 + b)
            start_scatter(b, c0 + b)
        for b in range(NBUF):
            wait_scatter(b, c0 + b)

    return k


def kernel(x, embed_weight, concepts):
    batch, hist = x.shape
    B = batch * hist
    num_embeds = embed_weight.shape[0]
    xw = x.reshape(NW, B // NW // CHUNK, CHUNK)
    out = _make_kernel(B, num_embeds)(xw, embed_weight, concepts)
    return out.reshape(batch, hist, DIM)


# speedup vs baseline: 7.4070x; 1.7559x over previous
import functools

import jax
import jax.numpy as jnp
from jax import lax
from jax.experimental import pallas as pl
from jax.experimental.pallas import tpu as pltpu
from jax.experimental.pallas import tpu_sc as plsc

DIM = 64
L = 16
NC = 2
NS = 16
NW = NC * NS
CHUNK = 128
NBUF = 8
NGRP = CHUNK // L


@functools.cache
def _make_kernel(B: int, num_embeds: int):
    b_per_w = B // NW
    n_chunks = b_per_w // CHUNK
    n_groups = n_chunks // NBUF
    assert n_chunks % NBUF == 0
    mesh = plsc.VectorSubcoreMesh(core_axis_name="c", subcore_axis_name="s")

    @functools.partial(
        pl.kernel,
        mesh=mesh,
        compiler_params=pltpu.CompilerParams(
            use_tc_tiling_on_sc=False, needs_layout_passes=False
        ),
        out_type=jax.ShapeDtypeStruct((B, 2 * DIM), jnp.float32),
        scratch_types=[
            pltpu.VMEM((n_chunks, CHUNK), jnp.int32),
            pltpu.VMEM((NBUF, CHUNK), jnp.int32),
            pltpu.VMEM((NBUF, CHUNK, DIM), jnp.float32),
            pltpu.VMEM((1, DIM), jnp.float32),
            pltpu.SemaphoreType.DMA((NBUF,)),
            pltpu.SemaphoreType.DMA((NBUF,)),
        ],
    )
    def k(x_hbm, table_hbm, conc_hbm, out_hbm, idx_v, cidx_v, rows_v, conc_v,
          gsem, ssem):
        wid = lax.axis_index("s") * NC + lax.axis_index("c")
        base = wid * b_per_w
        pltpu.sync_copy(conc_hbm, conc_v)
        pltpu.sync_copy(x_hbm.at[wid], idx_v)

        def clamp(b, c):
            for j in range(NGRP):
                v = idx_v[c, pl.ds(j * L, L)]
                cidx_v[b, pl.ds(j * L, L)] = jnp.where(v == num_embeds, 0, v)

        def fixup(b, c):
            acc = jnp.zeros((L,), jnp.int32)
            for j in range(NGRP):
                v = idx_v[c, pl.ds(j * L, L)]
                acc = acc + jnp.where(v == num_embeds, 1, 0)
            cnt = jnp.sum(acc)

            @pl.when(cnt > 0)
            def _():
                def grp_body(j, carry):
                    v = idx_v[c, pl.ds(j * L, L)]
                    mi = jnp.where(v == num_embeds, 1, 0)
                    gcnt = jnp.sum(mi)

                    @pl.when(gcnt > 0)
                    def _():
                        for l in range(L):
                            @pl.when(mi[l] > 0)
                            def _():
                                row = j * L + l
                                for q in range(DIM // L):
                                    rows_v[b, row, pl.ds(q * L, L)] = (
                                        conc_v[0, pl.ds(q * L, L)]
                                    )
                    return carry

                lax.fori_loop(0, NGRP, grp_body, 0)

        def start_gather(b):
            pltpu.async_copy(table_hbm.at[cidx_v.at[b]], rows_v.at[b], gsem.at[b])

        def wait_gather(b):
            pltpu.make_async_copy(
                table_hbm.at[cidx_v.at[b]], rows_v.at[b], gsem.at[b]
            ).wait()

        def start_scatter(b, c):
            off = pl.multiple_of(base + c * CHUNK, CHUNK)
            pltpu.async_copy(
                rows_v.at[b],
                out_hbm.at[pl.ds(off, CHUNK), pl.ds(0, DIM)],
                ssem.at[b],
            )

        def wait_scatter(b, c):
            off = pl.multiple_of(base + c * CHUNK, CHUNK)
            pltpu.make_async_copy(
                rows_v.at[b],
                out_hbm.at[pl.ds(off, CHUNK), pl.ds(0, DIM)],
                ssem.at[b],
            ).wait()

        for b in range(NBUF):
            clamp(b, b)
            start_gather(b)

        def group_body(g, carry):
            c0 = g * NBUF
            for b in range(NBUF):
                wait_gather(b)
                fixup(b, c0 + b)
                start_scatter(b, c0 + b)
            for b in range(NBUF):
                wait_scatter(b, c0 + b)
                clamp(b, c0 + NBUF + b)
                start_gather(b)
            return carry

        lax.fori_loop(0, n_groups - 1, group_body, 0)

        c0 = (n_groups - 1) * NBUF
        for b in range(NBUF):
            wait_gather(b)
            fixup(b, c0 + b)
            start_scatter(b, c0 + b)
        for b in range(NBUF):
            wait_scatter(b, c0 + b)

    return k


def kernel(x, embed_weight, concepts):
    batch, hist = x.shape
    B = batch * hist
    num_embeds = embed_weight.shape[0]
    xw = x.reshape(NW, B // NW // CHUNK, CHUNK)
    out = _make_kernel(B, num_embeds)(xw, embed_weight, concepts)
    return out.reshape(batch, hist, 2 * DIM)[:, :, :DIM]


# speedup vs baseline: 7.4146x; 1.0010x over previous
import functools

import jax
import jax.numpy as jnp
from jax import lax
from jax.experimental import pallas as pl
from jax.experimental.pallas import tpu as pltpu
from jax.experimental.pallas import tpu_sc as plsc

DIM = 64
L = 16
NC = 2
NS = 16
NW = NC * NS
CHUNK = 128
NBUF = 8
NGRP = CHUNK // L


@functools.cache
def _make_kernel(B: int, num_embeds: int):
    b_per_w = B // NW
    n_chunks = b_per_w // CHUNK
    n_groups = n_chunks // NBUF
    assert n_chunks % NBUF == 0
    mesh = plsc.VectorSubcoreMesh(core_axis_name="c", subcore_axis_name="s")

    @functools.partial(
        pl.kernel,
        mesh=mesh,
        compiler_params=pltpu.CompilerParams(
            use_tc_tiling_on_sc=False, needs_layout_passes=False
        ),
        out_type=jax.ShapeDtypeStruct((B, 2 * DIM), jnp.float32),
        scratch_types=[
            pltpu.VMEM((n_chunks, CHUNK), jnp.int32),
            pltpu.VMEM((NBUF, CHUNK), jnp.int32),
            pltpu.VMEM((NBUF, CHUNK, DIM), jnp.float32),
            pltpu.VMEM((1, DIM), jnp.float32),
            pltpu.SemaphoreType.DMA((NBUF,)),
            pltpu.SemaphoreType.DMA((NBUF,)),
        ],
    )
    def k(table_hbm, x_hbm, conc_hbm, out_hbm, idx_v, cidx_v, rows_v, conc_v,
          gsem, ssem):
        wid = lax.axis_index("s") * NC + lax.axis_index("c")
        base = wid * b_per_w
        pltpu.sync_copy(conc_hbm, conc_v)
        pltpu.sync_copy(x_hbm.at[wid], idx_v)

        def clamp(b, c):
            for j in range(NGRP):
                v = idx_v[c, pl.ds(j * L, L)]
                cidx_v[b, pl.ds(j * L, L)] = jnp.where(v == num_embeds, 0, v)

        def fixup(b, c):
            acc = jnp.zeros((L,), jnp.int32)
            for j in range(NGRP):
                v = idx_v[c, pl.ds(j * L, L)]
                acc = acc + jnp.where(v == num_embeds, 1, 0)
            cnt = jnp.sum(acc)

            @pl.when(cnt > 0)
            def _():
                def grp_body(j, carry):
                    v = idx_v[c, pl.ds(j * L, L)]
                    mi = jnp.where(v == num_embeds, 1, 0)
                    gcnt = jnp.sum(mi)

                    @pl.when(gcnt > 0)
                    def _():
                        for l in range(L):
                            @pl.when(mi[l] > 0)
                            def _():
                                row = j * L + l
                                for q in range(DIM // L):
                                    rows_v[b, row, pl.ds(q * L, L)] = (
                                        conc_v[0, pl.ds(q * L, L)]
                                    )
                    return carry

                lax.fori_loop(0, NGRP, grp_body, 0)

        def start_gather(b):
            pltpu.async_copy(table_hbm.at[cidx_v.at[b]], rows_v.at[b], gsem.at[b])

        def wait_gather(b):
            pltpu.make_async_copy(
                table_hbm.at[cidx_v.at[b]], rows_v.at[b], gsem.at[b]
            ).wait()

        def start_scatter(b, c):
            off = pl.multiple_of(base + c * CHUNK, CHUNK)
            pltpu.async_copy(
                rows_v.at[b],
                out_hbm.at[pl.ds(off, CHUNK), pl.ds(0, DIM)],
                ssem.at[b],
            )

        def wait_scatter(b, c):
            off = pl.multiple_of(base + c * CHUNK, CHUNK)
            pltpu.make_async_copy(
                rows_v.at[b],
                out_hbm.at[pl.ds(off, CHUNK), pl.ds(0, DIM)],
                ssem.at[b],
            ).wait()

        for b in range(NBUF):
            clamp(b, b)
            start_gather(b)

        def group_body(g, carry):
            c0 = g * NBUF
            for b in range(NBUF):
                wait_gather(b)
                fixup(b, c0 + b)
                start_scatter(b, c0 + b)
            for b in range(NBUF):
                wait_scatter(b, c0 + b)
                clamp(b, c0 + NBUF + b)
                start_gather(b)
            return carry

        lax.fori_loop(0, n_groups - 1, group_body, 0)

        c0 = (n_groups - 1) * NBUF
        for b in range(NBUF):
            wait_gather(b)
            fixup(b, c0 + b)
            start_scatter(b, c0 + b)
        for b in range(NBUF):
            wait_scatter(b, c0 + b)

    return k


def kernel(x, embed_weight, concepts):
    batch, hist = x.shape
    B = batch * hist
    num_embeds = embed_weight.shape[0]
    xw = x.reshape(NW, B // NW // CHUNK, CHUNK)
    out = _make_kernel(B, num_embeds)(embed_weight, xw, concepts)
    return out.reshape(batch, hist, 2 * DIM)[:, :, :DIM]


# speedup vs baseline: 7.4192x; 1.0006x over previous
import functools

import jax
import jax.numpy as jnp
from jax import lax
from jax.experimental import pallas as pl
from jax.experimental.pallas import tpu as pltpu
from jax.experimental.pallas import tpu_sc as plsc

DIM = 64
L = 16
NC = 2
NS = 16
NW = NC * NS
CHUNK = 128
NBUF = 8
NGRP = CHUNK // L


@functools.cache
def _make_kernel(batch: int, hist: int, num_embeds: int):
    n_groups = hist // NBUF
    assert batch % (NW * CHUNK) == 0 and batch // NW == CHUNK
    mesh = plsc.VectorSubcoreMesh(core_axis_name="c", subcore_axis_name="s")

    @functools.partial(
        pl.kernel,
        mesh=mesh,
        compiler_params=pltpu.CompilerParams(
            use_tc_tiling_on_sc=False, needs_layout_passes=False
        ),
        out_type=jax.ShapeDtypeStruct((NW, CHUNK, hist, 2 * DIM), jnp.float32),
        scratch_types=[
            pltpu.VMEM((n_groups, NBUF, CHUNK), jnp.int32),
            pltpu.VMEM((NBUF, CHUNK), jnp.int32),
            pltpu.VMEM((NBUF, CHUNK, DIM), jnp.float32),
            pltpu.VMEM((1, DIM), jnp.float32),
            pltpu.SemaphoreType.DMA((NBUF,)),
            pltpu.SemaphoreType.DMA((NBUF,)),
        ],
    )
    def k(table_hbm, x4_hbm, conc_hbm, out_hbm, idx_v, cidx_v, rows_v, conc_v,
          gsem, ssem):
        wid = lax.axis_index("s") * NC + lax.axis_index("c")
        pltpu.sync_copy(conc_hbm, conc_v)
        pltpu.sync_copy(x4_hbm.at[:, wid], idx_v)

        def clamp(b, g):
            for j in range(NGRP):
                v = idx_v[g, b, pl.ds(j * L, L)]
                cidx_v[b, pl.ds(j * L, L)] = jnp.where(v == num_embeds, 0, v)

        def fixup(b, g):
            acc = jnp.zeros((L,), jnp.int32)
            for j in range(NGRP):
                v = idx_v[g, b, pl.ds(j * L, L)]
                acc = acc + jnp.where(v == num_embeds, 1, 0)
            cnt = jnp.sum(acc)

            @pl.when(cnt > 0)
            def _():
                def grp_body(j, carry):
                    v = idx_v[g, b, pl.ds(j * L, L)]
                    mi = jnp.where(v == num_embeds, 1, 0)
                    gcnt = jnp.sum(mi)

                    @pl.when(gcnt > 0)
                    def _():
                        for l in range(L):
                            @pl.when(mi[l] > 0)
                            def _():
                                row = j * L + l
                                for q in range(DIM // L):
                                    rows_v[b, row, pl.ds(q * L, L)] = (
                                        conc_v[0, pl.ds(q * L, L)]
                                    )
                    return carry

                lax.fori_loop(0, NGRP, grp_body, 0)

        def start_gather(b):
            pltpu.async_copy(table_hbm.at[cidx_v.at[b]], rows_v.at[b], gsem.at[b])

        def wait_gather(b):
            pltpu.make_async_copy(
                table_hbm.at[cidx_v.at[b]], rows_v.at[b], gsem.at[b]
            ).wait()

        def _scatter_dst(b, g):
            h = g * NBUF + b
            return out_hbm.at[wid, pl.ds(0, CHUNK), h, pl.ds(0, DIM)]

        def start_scatter(b, g):
            pltpu.async_copy(rows_v.at[b], _scatter_dst(b, g), ssem.at[b])

        def wait_scatter(b, g):
            pltpu.make_async_copy(rows_v.at[b], _scatter_dst(b, g), ssem.at[b]).wait()

        for b in range(NBUF):
            clamp(b, 0)
            start_gather(b)

        def group_body(g, carry):
            for b in range(NBUF):
                wait_gather(b)
                fixup(b, g)
                start_scatter(b, g)
            for b in range(NBUF):
                wait_scatter(b, g)
                clamp(b, g + 1)
                start_gather(b)
            return carry

        lax.fori_loop(0, n_groups - 1, group_body, 0)

        g_last = n_groups - 1
        for b in range(NBUF):
            wait_gather(b)
            fixup(b, g_last)
            start_scatter(b, g_last)
        for b in range(NBUF):
            wait_scatter(b, g_last)

    return k


def kernel(x, embed_weight, concepts):
    batch, hist = x.shape
    num_embeds = embed_weight.shape[0]
    x4 = x.T.reshape(hist // NBUF, NBUF, NW, CHUNK).transpose(0, 2, 1, 3)
    out = _make_kernel(batch, hist, num_embeds)(embed_weight, x4, concepts)
    return out.reshape(batch, hist, 2 * DIM)[:, :, :DIM]
